# Initial kernel scaffold; baseline (speedup 1.0000x reference)
import jax
import jax.numpy as jnp
from jax.experimental import pallas as pl


def kernel(x, edge_index, W_self_0, W_neigh_0, b_0, W_self_1, W_neigh_1, b_1):
    raise NotImplementedError("write your pallas kernel here")


# speedup vs baseline: 2.9624x; 2.9624x over previous
import functools

import jax
import jax.numpy as jnp
from jax import lax
from jax.experimental import pallas as pl
from jax.experimental.pallas import tpu as pltpu
from jax.experimental.pallas import tpu_sc as plsc

N = 10000
D = 128
E = 320000

NC = 2
NS = 16
NW = NC * NS
L = 16
NP = 10112
RPT = NP // NS
ZB = 79
EW = E // NW
PKW = EW // 2
PKP = 5120
PADE = (PKP - PKW) * 2
B = 64
ROWS = PKP // 128
CPR = 4

DRT = 640

RB = 1000
GRID = N // RB

_f32 = jnp.float32


def _make_sc_agg(with_deg: bool):
  mesh = plsc.VectorSubcoreMesh(
      core_axis_name="c", subcore_axis_name="s",
      num_cores=NC, num_subcores=NS)
  out_type = [jax.ShapeDtypeStruct((NC, NP, D), _f32)]
  scratch = [
      pltpu.VMEM((ROWS, 128), jnp.int32),
      pltpu.VMEM((ROWS, 128), jnp.int32),
      pltpu.VMEM((L, D), _f32),
      pltpu.VMEM((ZB, D), _f32),
      pltpu.VMEM((8, D), _f32),
      pltpu.VMEM_SHARED((NP, D), _f32),
      pltpu.SemaphoreType.DMA,
  ]
  if with_deg:
    out_type.append(jax.ShapeDtypeStruct((NC * NS * DRT,), _f32))
    scratch += [
        pltpu.VMEM((L,), _f32),
        pltpu.VMEM((DRT,), _f32),
        pltpu.VMEM_SHARED((NS * DRT,), _f32),
    ]

  def body(z_hbm, src_hbm, dst_hbm, *rest):
    if with_deg:
      (agg_out, deg_out, srcp_v, dstp_v,
       rows_v, zbuf, wb8, agg_sh, sem, ones_v, degb, deg_sh) = rest
    else:
      (agg_out, srcp_v, dstp_v, rows_v, zbuf, wb8, agg_sh, sem) = rest

    cid = lax.axis_index("c")
    sid = lax.axis_index("s")
    w = cid * NS + sid
    base = sid * RPT
    zero16 = jnp.zeros((L,), _f32)

    def zb(i, c):
      for k in range(D // L):
        zbuf[i, pl.ds(k * L, L)] = zero16
      return c
    lax.fori_loop(0, ZB, zb, 0)

    for t in range(RPT // ZB):
      pltpu.sync_copy(zbuf, agg_sh.at[pl.ds(base + t * ZB, ZB)])
    pltpu.sync_copy(src_hbm.at[w], srcp_v)
    pltpu.sync_copy(dst_hbm.at[w], dstp_v)

    if with_deg:
      def zdg(i, c):
        degb[pl.ds(i * L, L)] = zero16
        return c
      lax.fori_loop(0, DRT // L, zdg, 0)
      pltpu.sync_copy(degb, deg_sh.at[pl.ds(sid * DRT, DRT)])
      ones_v[pl.ds(0, L)] = jnp.ones((L,), _f32)

    plsc.subcore_barrier()

    def row(rr, c):
      for g in range(8):
        spk = srcp_v[rr, pl.ds(g * L, L)]
        dpk = dstp_v[rr, pl.ds(g * L, L)]
        for sv, dv in (
            (spk & 0xFFFF, dpk & 0xFFFF),
            (lax.shift_right_logical(spk, 16),
             lax.shift_right_logical(dpk, 16)),
        ):
          pltpu.async_copy(z_hbm.at[sv], rows_v, sem).wait()
          pltpu.sync_copy(rows_v, agg_sh.at[dv], add=True)
          if with_deg:
            pltpu.sync_copy(ones_v, deg_sh.at[dv], add=True)
      return c
    lax.fori_loop(0, ROWS, row, 0)

    plsc.subcore_barrier()

    def wb(t, c):
      pltpu.sync_copy(agg_sh.at[pl.ds(base + t * 8, 8)], wb8)
      pltpu.sync_copy(wb8, agg_out.at[cid, pl.ds(base + t * 8, 8)])
      return c
    lax.fori_loop(0, RPT // 8, wb, 0)
    if with_deg:
      pltpu.sync_copy(deg_sh.at[pl.ds(sid * DRT, DRT)], degb)
      pltpu.sync_copy(degb, deg_out.at[pl.ds((cid * NS + sid) * DRT, DRT)])

  return pl.kernel(body, out_type=out_type, mesh=mesh,
                   scratch_types=scratch)


_sc_agg_deg = _make_sc_agg(True)
_sc_agg = _make_sc_agg(False)


def _first_body(x_ref, wn_ref, ws_ref, b_ref, z_ref, s_ref):
  xb = x_ref[...]
  z_ref[...] = jnp.dot(xb, wn_ref[...], preferred_element_type=_f32)
  s_ref[...] = jnp.dot(xb, ws_ref[...], preferred_element_type=_f32) + b_ref[...]


def _mid_body(s0_ref, agg_ref, deg_ref, wn_ref, ws_ref, b_ref, z_ref, s_ref):
  a = agg_ref[...]
  d = deg_ref[...]
  inv = 1.0 / jnp.maximum(d[0] + d[1], 1.0)
  h = jnp.maximum(s0_ref[...] + (a[0] + a[1]) * inv, 0.0)
  z_ref[...] = jnp.dot(h, wn_ref[...], preferred_element_type=_f32)
  s_ref[...] = jnp.dot(h, ws_ref[...], preferred_element_type=_f32) + b_ref[...]


def _fin_body(s1_ref, agg_ref, deg_ref, o_ref):
  a = agg_ref[...]
  d = deg_ref[...]
  inv = 1.0 / jnp.maximum(d[0] + d[1], 1.0)
  o_ref[...] = s1_ref[...] + (a[0] + a[1]) * inv


_row_spec = pl.BlockSpec((RB, D), lambda i: (i, 0))
_agg_spec = pl.BlockSpec((NC, RB, D), lambda i: (0, i, 0))
_deg_spec = pl.BlockSpec((NC, RB, 1), lambda i: (0, i, 0))
_w_spec = pl.BlockSpec((D, D), lambda i: (0, 0))
_b_spec = pl.BlockSpec((1, D), lambda i: (0, 0))
_nd_shape = jax.ShapeDtypeStruct((N, D), _f32)

_mm_first = pl.pallas_call(
    _first_body, grid=(GRID,),
    in_specs=[_row_spec, _w_spec, _w_spec, _b_spec],
    out_specs=[_row_spec, _row_spec],
    out_shape=[_nd_shape, _nd_shape])

_mm_mid = pl.pallas_call(
    _mid_body, grid=(GRID,),
    in_specs=[_row_spec, _agg_spec, _deg_spec, _w_spec, _w_spec, _b_spec],
    out_specs=[_row_spec, _row_spec],
    out_shape=[_nd_shape, _nd_shape])

_mm_fin = pl.pallas_call(
    _fin_body, grid=(GRID,),
    in_specs=[_row_spec, _agg_spec, _deg_spec],
    out_specs=_row_spec,
    out_shape=_nd_shape)


def _pack(idx, pad_vals):
  pk = idx[0::2] | (idx[1::2] << 16)
  pk = pk.reshape(NW, PKW)
  pad = jnp.broadcast_to(pad_vals[None, :], (NW, PKP - PKW))
  return jnp.concatenate([pk, pad], axis=1).reshape(NW, ROWS, 128)


def kernel(x, edge_index, W_self_0, W_neigh_0, b_0, W_self_1, W_neigh_1, b_1):
  spad = (jnp.arange(PKP - PKW, dtype=jnp.int32) * 79) % N
  spad = spad | (((spad * 3 + 11) % N) << 16)
  dpad = N + (jnp.arange(PKP - PKW, dtype=jnp.int32) & 63)
  dpad = dpad | (dpad << 16)
  src = _pack(edge_index[0], spad)
  dst = _pack(edge_index[1], dpad)
  b0 = b_0.reshape(1, D)
  b1 = b_1.reshape(1, D)

  z0, s0 = _mm_first(x, W_neigh_0, W_self_0, b0)
  aggp, degf = _sc_agg_deg(z0, src, dst)
  degp = degf.reshape(NC, NS * DRT)[:, :N].reshape(NC, N, 1)
  z1, s1 = _mm_mid(s0, aggp, degp, W_neigh_1, W_self_1, b1)
  (aggp1,) = _sc_agg(z1, src, dst)
  return _mm_fin(s1, aggp1, degp)


# speedup vs baseline: 5.8008x; 1.9581x over previous
import functools

import jax
import jax.numpy as jnp
from jax import lax
from jax.experimental import pallas as pl
from jax.experimental.pallas import tpu as pltpu
from jax.experimental.pallas import tpu_sc as plsc

N = 10000
D = 128
E = 320000

NC = 2
NS = 16
NW = NC * NS
L = 16
NP = 10112
RPT = NP // NS
ZB = 79
EW = E // NW
PKW = EW // 2
PKP = 5120
PADE = (PKP - PKW) * 2
B = 64
ROWS = PKP // 128
CPR = 4

DRT = 640

RB = 1000
GRID = N // RB

_f32 = jnp.float32


def _make_sc_agg(with_deg: bool):
  mesh = plsc.VectorSubcoreMesh(
      core_axis_name="c", subcore_axis_name="s",
      num_cores=NC, num_subcores=NS)
  out_type = [jax.ShapeDtypeStruct((NC, NP, D), _f32)]
  scratch = [
      pltpu.VMEM((ROWS, 128), jnp.int32),
      pltpu.VMEM((ROWS, 128), jnp.int32),
      pltpu.VMEM((8, L, D), _f32),
      pltpu.VMEM((ZB, D), _f32),
      pltpu.VMEM((8, D), _f32),
      pltpu.VMEM_SHARED((NP, D), _f32),
      pltpu.SemaphoreType.DMA,
  ]
  if with_deg:
    out_type.append(jax.ShapeDtypeStruct((NC * NS * DRT,), _f32))
    scratch += [
        pltpu.VMEM((L,), _f32),
        pltpu.VMEM((DRT,), _f32),
        pltpu.VMEM_SHARED((NS * DRT,), _f32),
    ]

  def body(z_hbm, src_hbm, dst_hbm, *rest):
    if with_deg:
      (agg_out, deg_out, srcp_v, dstp_v,
       rows_v, zbuf, wb8, agg_sh, sem, ones_v, degb, deg_sh) = rest
    else:
      (agg_out, srcp_v, dstp_v, rows_v, zbuf, wb8, agg_sh, sem) = rest

    cid = lax.axis_index("c")
    sid = lax.axis_index("s")
    w = cid * NS + sid
    base = sid * RPT
    zero16 = jnp.zeros((L,), _f32)

    def zb(i, c):
      for k in range(D // L):
        zbuf[i, pl.ds(k * L, L)] = zero16
      return c
    lax.fori_loop(0, ZB, zb, 0)

    for t in range(RPT // ZB):
      pltpu.sync_copy(zbuf, agg_sh.at[pl.ds(base + t * ZB, ZB)])
    pltpu.sync_copy(src_hbm.at[w], srcp_v)
    pltpu.sync_copy(dst_hbm.at[w], dstp_v)

    if with_deg:
      def zdg(i, c):
        degb[pl.ds(i * L, L)] = zero16
        return c
      lax.fori_loop(0, DRT // L, zdg, 0)
      pltpu.sync_copy(degb, deg_sh.at[pl.ds(sid * DRT, DRT)])
      ones_v[pl.ds(0, L)] = jnp.ones((L,), _f32)

    plsc.subcore_barrier()

    def row(rr, c):
      for ph in range(2):
        svs, dvs = [], []
        for g in range(4):
          spk = srcp_v[rr, pl.ds((ph * 4 + g) * L, L)]
          dpk = dstp_v[rr, pl.ds((ph * 4 + g) * L, L)]
          svs += [spk & 0xFFFF, lax.shift_right_logical(spk, 16)]
          dvs += [dpk & 0xFFFF, lax.shift_right_logical(dpk, 16)]
        copies = [pltpu.async_copy(z_hbm.at[svs[i]], rows_v.at[i], sem)
                  for i in range(8)]
        for i in range(8):
          copies[i].wait()
        for i in range(8):
          pltpu.sync_copy(rows_v.at[i], agg_sh.at[dvs[i]], add=True)
          if with_deg:
            pltpu.sync_copy(ones_v, deg_sh.at[dvs[i]], add=True)
      return c
    lax.fori_loop(0, ROWS, row, 0)

    plsc.subcore_barrier()

    def wb(t, c):
      pltpu.sync_copy(agg_sh.at[pl.ds(base + t * 8, 8)], wb8)
      pltpu.sync_copy(wb8, agg_out.at[cid, pl.ds(base + t * 8, 8)])
      return c
    lax.fori_loop(0, RPT // 8, wb, 0)
    if with_deg:
      pltpu.sync_copy(deg_sh.at[pl.ds(sid * DRT, DRT)], degb)
      pltpu.sync_copy(degb, deg_out.at[pl.ds((cid * NS + sid) * DRT, DRT)])

  return pl.kernel(body, out_type=out_type, mesh=mesh,
                   scratch_types=scratch)


_sc_agg_deg = _make_sc_agg(True)
_sc_agg = _make_sc_agg(False)


def _first_body(x_ref, wn_ref, ws_ref, b_ref, z_ref, s_ref):
  xb = x_ref[...]
  z_ref[...] = jnp.dot(xb, wn_ref[...], preferred_element_type=_f32)
  s_ref[...] = jnp.dot(xb, ws_ref[...], preferred_element_type=_f32) + b_ref[...]


def _mid_body(s0_ref, agg_ref, deg_ref, wn_ref, ws_ref, b_ref, z_ref, s_ref):
  a = agg_ref[...]
  d = deg_ref[...]
  inv = 1.0 / jnp.maximum(d[0] + d[1], 1.0)
  h = jnp.maximum(s0_ref[...] + (a[0] + a[1]) * inv, 0.0)
  z_ref[...] = jnp.dot(h, wn_ref[...], preferred_element_type=_f32)
  s_ref[...] = jnp.dot(h, ws_ref[...], preferred_element_type=_f32) + b_ref[...]


def _fin_body(s1_ref, agg_ref, deg_ref, o_ref):
  a = agg_ref[...]
  d = deg_ref[...]
  inv = 1.0 / jnp.maximum(d[0] + d[1], 1.0)
  o_ref[...] = s1_ref[...] + (a[0] + a[1]) * inv


_row_spec = pl.BlockSpec((RB, D), lambda i: (i, 0))
_agg_spec = pl.BlockSpec((NC, RB, D), lambda i: (0, i, 0))
_deg_spec = pl.BlockSpec((NC, RB, 1), lambda i: (0, i, 0))
_w_spec = pl.BlockSpec((D, D), lambda i: (0, 0))
_b_spec = pl.BlockSpec((1, D), lambda i: (0, 0))
_nd_shape = jax.ShapeDtypeStruct((N, D), _f32)

_mm_first = pl.pallas_call(
    _first_body, grid=(GRID,),
    in_specs=[_row_spec, _w_spec, _w_spec, _b_spec],
    out_specs=[_row_spec, _row_spec],
    out_shape=[_nd_shape, _nd_shape])

_mm_mid = pl.pallas_call(
    _mid_body, grid=(GRID,),
    in_specs=[_row_spec, _agg_spec, _deg_spec, _w_spec, _w_spec, _b_spec],
    out_specs=[_row_spec, _row_spec],
    out_shape=[_nd_shape, _nd_shape])

_mm_fin = pl.pallas_call(
    _fin_body, grid=(GRID,),
    in_specs=[_row_spec, _agg_spec, _deg_spec],
    out_specs=_row_spec,
    out_shape=_nd_shape)


def _pack(idx, pad_vals):
  pk = idx[0::2] | (idx[1::2] << 16)
  pk = pk.reshape(NW, PKW)
  pad = jnp.broadcast_to(pad_vals[None, :], (NW, PKP - PKW))
  return jnp.concatenate([pk, pad], axis=1).reshape(NW, ROWS, 128)


def kernel(x, edge_index, W_self_0, W_neigh_0, b_0, W_self_1, W_neigh_1, b_1):
  spad = (jnp.arange(PKP - PKW, dtype=jnp.int32) * 79) % N
  spad = spad | (((spad * 3 + 11) % N) << 16)
  dpad = N + (jnp.arange(PKP - PKW, dtype=jnp.int32) & 63)
  dpad = dpad | (dpad << 16)
  src = _pack(edge_index[0], spad)
  dst = _pack(edge_index[1], dpad)
  b0 = b_0.reshape(1, D)
  b1 = b_1.reshape(1, D)

  z0, s0 = _mm_first(x, W_neigh_0, W_self_0, b0)
  aggp, degf = _sc_agg_deg(z0, src, dst)
  degp = degf.reshape(NC, NS * DRT)[:, :N].reshape(NC, N, 1)
  z1, s1 = _mm_mid(s0, aggp, degp, W_neigh_1, W_self_1, b1)
  (aggp1,) = _sc_agg(z1, src, dst)
  return _mm_fin(s1, aggp1, degp)


# speedup vs baseline: 7.2783x; 1.2547x over previous
import functools

import jax
import jax.numpy as jnp
from jax import lax
from jax.experimental import pallas as pl
from jax.experimental.pallas import tpu as pltpu
from jax.experimental.pallas import tpu_sc as plsc

N = 10000
D = 128
E = 320000

NC = 2
NS = 16
NW = NC * NS
L = 16
NP = 10112
RPT = NP // NS
ZB = 79
EW = E // NW
PKW = EW // 2
PKP = 5120
PADE = (PKP - PKW) * 2
B = 64
ROWS = PKP // 128
CPR = 4

DRT = 640

RB = 1000
GRID = N // RB

_f32 = jnp.float32


def _make_sc_agg(with_deg: bool):
  mesh = plsc.VectorSubcoreMesh(
      core_axis_name="c", subcore_axis_name="s",
      num_cores=NC, num_subcores=NS)
  out_type = [jax.ShapeDtypeStruct((NC, NP, D), _f32)]
  scratch = [
      pltpu.VMEM((ROWS, 128), jnp.int32),
      pltpu.VMEM((ROWS, 128), jnp.int32),
      pltpu.VMEM((8, L, D), _f32),
      pltpu.VMEM((ZB, D), _f32),
      pltpu.VMEM((8, D), _f32),
      pltpu.VMEM_SHARED((NP, D), _f32),
      pltpu.SemaphoreType.DMA((8,)),
      pltpu.SemaphoreType.DMA((8,)),
      pltpu.SemaphoreType.DMA,
  ]
  if with_deg:
    out_type.append(jax.ShapeDtypeStruct((NC * NS * DRT,), _f32))
    scratch += [
        pltpu.VMEM((L,), _f32),
        pltpu.VMEM((DRT,), _f32),
        pltpu.VMEM_SHARED((NS * DRT,), _f32),
    ]

  def body(z_hbm, src_hbm, dst_hbm, *rest):
    if with_deg:
      (agg_out, deg_out, srcp_v, dstp_v, rows_v, zbuf, wb8, agg_sh,
       semg, sems, semd, ones_v, degb, deg_sh) = rest
    else:
      (agg_out, srcp_v, dstp_v, rows_v, zbuf, wb8, agg_sh,
       semg, sems, semd) = rest

    cid = lax.axis_index("c")
    sid = lax.axis_index("s")
    w = cid * NS + sid
    base = sid * RPT
    zero16 = jnp.zeros((L,), _f32)

    def zb(i, c):
      for k in range(D // L):
        zbuf[i, pl.ds(k * L, L)] = zero16
      return c
    lax.fori_loop(0, ZB, zb, 0)

    for t in range(RPT // ZB):
      pltpu.sync_copy(zbuf, agg_sh.at[pl.ds(base + t * ZB, ZB)])
    pltpu.sync_copy(src_hbm.at[w], srcp_v)
    pltpu.sync_copy(dst_hbm.at[w], dstp_v)

    if with_deg:
      def zdg(i, c):
        degb[pl.ds(i * L, L)] = zero16
        return c
      lax.fori_loop(0, DRT // L, zdg, 0)
      pltpu.sync_copy(degb, deg_sh.at[pl.ds(sid * DRT, DRT)])
      ones_v[pl.ds(0, L)] = jnp.ones((L,), _f32)

    plsc.subcore_barrier()

    def row(rr, c):
      for ph in range(2):
        svs, dvs = [], []
        for g in range(4):
          spk = srcp_v[rr, pl.ds((ph * 4 + g) * L, L)]
          dpk = dstp_v[rr, pl.ds((ph * 4 + g) * L, L)]
          svs += [spk & 0xFFFF, lax.shift_right_logical(spk, 16)]
          dvs += [dpk & 0xFFFF, lax.shift_right_logical(dpk, 16)]
        copies = [pltpu.async_copy(z_hbm.at[svs[i]], rows_v.at[i],
                                   semg.at[i]) for i in range(8)]
        scs, dgs = [], []
        for i in range(8):
          copies[i].wait()
          scs.append(pltpu.async_copy(rows_v.at[i], agg_sh.at[dvs[i]],
                                      sems.at[i], add=True))
          if with_deg:
            dgs.append(pltpu.async_copy(ones_v, deg_sh.at[dvs[i]],
                                        semd, add=True))
        for d in scs + dgs:
          d.wait()
      return c
    lax.fori_loop(0, ROWS, row, 0)

    plsc.subcore_barrier()

    def wb(t, c):
      pltpu.sync_copy(agg_sh.at[pl.ds(base + t * 8, 8)], wb8)
      pltpu.sync_copy(wb8, agg_out.at[cid, pl.ds(base + t * 8, 8)])
      return c
    lax.fori_loop(0, RPT // 8, wb, 0)
    if with_deg:
      pltpu.sync_copy(deg_sh.at[pl.ds(sid * DRT, DRT)], degb)
      pltpu.sync_copy(degb, deg_out.at[pl.ds((cid * NS + sid) * DRT, DRT)])

  return pl.kernel(body, out_type=out_type, mesh=mesh,
                   scratch_types=scratch)


_sc_agg_deg = _make_sc_agg(True)
_sc_agg = _make_sc_agg(False)


def _first_body(x_ref, wn_ref, ws_ref, b_ref, z_ref, s_ref):
  xb = x_ref[...]
  z_ref[...] = jnp.dot(xb, wn_ref[...], preferred_element_type=_f32)
  s_ref[...] = jnp.dot(xb, ws_ref[...], preferred_element_type=_f32) + b_ref[...]


def _mid_body(s0_ref, agg_ref, deg_ref, wn_ref, ws_ref, b_ref, z_ref, s_ref):
  a = agg_ref[...]
  d = deg_ref[...]
  inv = 1.0 / jnp.maximum(d[0] + d[1], 1.0)
  h = jnp.maximum(s0_ref[...] + (a[0] + a[1]) * inv, 0.0)
  z_ref[...] = jnp.dot(h, wn_ref[...], preferred_element_type=_f32)
  s_ref[...] = jnp.dot(h, ws_ref[...], preferred_element_type=_f32) + b_ref[...]


def _fin_body(s1_ref, agg_ref, deg_ref, o_ref):
  a = agg_ref[...]
  d = deg_ref[...]
  inv = 1.0 / jnp.maximum(d[0] + d[1], 1.0)
  o_ref[...] = s1_ref[...] + (a[0] + a[1]) * inv


_row_spec = pl.BlockSpec((RB, D), lambda i: (i, 0))
_agg_spec = pl.BlockSpec((NC, RB, D), lambda i: (0, i, 0))
_deg_spec = pl.BlockSpec((NC, RB, 1), lambda i: (0, i, 0))
_w_spec = pl.BlockSpec((D, D), lambda i: (0, 0))
_b_spec = pl.BlockSpec((1, D), lambda i: (0, 0))
_nd_shape = jax.ShapeDtypeStruct((N, D), _f32)

_mm_first = pl.pallas_call(
    _first_body, grid=(GRID,),
    in_specs=[_row_spec, _w_spec, _w_spec, _b_spec],
    out_specs=[_row_spec, _row_spec],
    out_shape=[_nd_shape, _nd_shape])

_mm_mid = pl.pallas_call(
    _mid_body, grid=(GRID,),
    in_specs=[_row_spec, _agg_spec, _deg_spec, _w_spec, _w_spec, _b_spec],
    out_specs=[_row_spec, _row_spec],
    out_shape=[_nd_shape, _nd_shape])

_mm_fin = pl.pallas_call(
    _fin_body, grid=(GRID,),
    in_specs=[_row_spec, _agg_spec, _deg_spec],
    out_specs=_row_spec,
    out_shape=_nd_shape)


def _pack(idx, pad_vals):
  pk = idx[0::2] | (idx[1::2] << 16)
  pk = pk.reshape(NW, PKW)
  pad = jnp.broadcast_to(pad_vals[None, :], (NW, PKP - PKW))
  return jnp.concatenate([pk, pad], axis=1).reshape(NW, ROWS, 128)


def kernel(x, edge_index, W_self_0, W_neigh_0, b_0, W_self_1, W_neigh_1, b_1):
  spad = (jnp.arange(PKP - PKW, dtype=jnp.int32) * 79) % N
  spad = spad | (((spad * 3 + 11) % N) << 16)
  dpad = N + (jnp.arange(PKP - PKW, dtype=jnp.int32) & 63)
  dpad = dpad | (dpad << 16)
  src = _pack(edge_index[0], spad)
  dst = _pack(edge_index[1], dpad)
  b0 = b_0.reshape(1, D)
  b1 = b_1.reshape(1, D)

  z0, s0 = _mm_first(x, W_neigh_0, W_self_0, b0)
  aggp, degf = _sc_agg_deg(z0, src, dst)
  degp = degf.reshape(NC, NS * DRT)[:, :N].reshape(NC, N, 1)
  z1, s1 = _mm_mid(s0, aggp, degp, W_neigh_1, W_self_1, b1)
  (aggp1,) = _sc_agg(z1, src, dst)
  return _mm_fin(s1, aggp1, degp)


# speedup vs baseline: 7.4426x; 1.0226x over previous
import functools

import jax
import jax.numpy as jnp
from jax import lax
from jax.experimental import pallas as pl
from jax.experimental.pallas import tpu as pltpu
from jax.experimental.pallas import tpu_sc as plsc

N = 10000
D = 128
E = 320000

NC = 2
NS = 16
NW = NC * NS
L = 16
NP = 10112
RPT = NP // NS
ZB = 79
EW = E // NW
PKW = EW // 2
PKP = 5120
PADE = (PKP - PKW) * 2
B = 64
ROWS = PKP // 128
CPR = 4

DRT = 640

RB = 1000
GRID = N // RB

_f32 = jnp.float32


def _make_sc_agg(with_deg: bool):
  mesh = plsc.VectorSubcoreMesh(
      core_axis_name="c", subcore_axis_name="s",
      num_cores=NC, num_subcores=NS)
  out_type = [jax.ShapeDtypeStruct((NC, NP, D), _f32)]
  scratch = [
      pltpu.VMEM((ROWS, 128), jnp.int32),
      pltpu.VMEM((ROWS, 128), jnp.int32),
      pltpu.VMEM((4, B), jnp.int32),
      pltpu.VMEM((4, B), jnp.int32),
      pltpu.VMEM((4, B, D), _f32),
      pltpu.VMEM((8, D), _f32),
      pltpu.VMEM_SHARED((NP, D), _f32),
      pltpu.SemaphoreType.DMA((4,)),
      pltpu.SemaphoreType.DMA((4,)),
      pltpu.SemaphoreType.DMA,
  ]
  if with_deg:
    out_type.append(jax.ShapeDtypeStruct((NC * NS * DRT,), _f32))
    scratch += [
        pltpu.VMEM((B,), _f32),
        pltpu.VMEM((DRT,), _f32),
        pltpu.VMEM_SHARED((NS * DRT,), _f32),
    ]

  def body(z_hbm, src_hbm, dst_hbm, *rest):
    if with_deg:
      (agg_out, deg_out, srcp_v, dstp_v, sidx, didx, rows_v, wb8,
       agg_sh, semg, sems, semd, ones_v, degb, deg_sh) = rest
    else:
      (agg_out, srcp_v, dstp_v, sidx, didx, rows_v, wb8, agg_sh,
       semg, sems, semd) = rest

    cid = lax.axis_index("c")
    sid = lax.axis_index("s")
    w = cid * NS + sid
    base = sid * RPT
    zero16 = jnp.zeros((L,), _f32)

    def zb(i, c):
      for k in range(D // L):
        wb8[i, pl.ds(k * L, L)] = zero16
      return c
    lax.fori_loop(0, 8, zb, 0)

    def zs(t, c):
      pltpu.sync_copy(wb8, agg_sh.at[pl.ds(base + t * 8, 8)])
      return c
    lax.fori_loop(0, RPT // 8, zs, 0)
    pltpu.sync_copy(src_hbm.at[w], srcp_v)
    pltpu.sync_copy(dst_hbm.at[w], dstp_v)

    if with_deg:
      def zdg(i, c):
        degb[pl.ds(i * L, L)] = zero16
        return c
      lax.fori_loop(0, DRT // L, zdg, 0)
      pltpu.sync_copy(degb, deg_sh.at[pl.ds(sid * DRT, DRT)])
      for k in range(B // L):
        ones_v[pl.ds(k * L, L)] = jnp.ones((L,), _f32)

    plsc.subcore_barrier()

    def row(rr, c):
      copies = []
      for q in range(4):
        for g in range(2):
          spk = srcp_v[rr, pl.ds((q * 2 + g) * L, L)]
          dpk = dstp_v[rr, pl.ds((q * 2 + g) * L, L)]
          sidx[q, pl.ds(g * 2 * L, L)] = spk & 0xFFFF
          sidx[q, pl.ds((g * 2 + 1) * L, L)] = lax.shift_right_logical(spk, 16)
          didx[q, pl.ds(g * 2 * L, L)] = dpk & 0xFFFF
          didx[q, pl.ds((g * 2 + 1) * L, L)] = lax.shift_right_logical(dpk, 16)
        copies.append(pltpu.async_copy(z_hbm.at[sidx.at[q]], rows_v.at[q],
                                       semg.at[q]))
      scs, dgs = [], []
      for q in range(4):
        copies[q].wait()
        scs.append(pltpu.async_copy(rows_v.at[q], agg_sh.at[didx.at[q]],
                                    sems.at[q], add=True))
        if with_deg:
          dgs.append(pltpu.async_copy(ones_v, deg_sh.at[didx.at[q]],
                                      semd, add=True))
      for d in scs + dgs:
        d.wait()
      return c
    lax.fori_loop(0, ROWS, row, 0)

    plsc.subcore_barrier()

    def wb(t, c):
      pltpu.sync_copy(agg_sh.at[pl.ds(base + t * 8, 8)], wb8)
      pltpu.sync_copy(wb8, agg_out.at[cid, pl.ds(base + t * 8, 8)])
      return c
    lax.fori_loop(0, RPT // 8, wb, 0)
    if with_deg:
      pltpu.sync_copy(deg_sh.at[pl.ds(sid * DRT, DRT)], degb)
      pltpu.sync_copy(degb, deg_out.at[pl.ds((cid * NS + sid) * DRT, DRT)])

  return pl.kernel(body, out_type=out_type, mesh=mesh,
                   scratch_types=scratch)


_sc_agg_deg = _make_sc_agg(True)
_sc_agg = _make_sc_agg(False)


def _first_body(x_ref, wn_ref, ws_ref, b_ref, z_ref, s_ref):
  xb = x_ref[...]
  z_ref[...] = jnp.dot(xb, wn_ref[...], preferred_element_type=_f32)
  s_ref[...] = jnp.dot(xb, ws_ref[...], preferred_element_type=_f32) + b_ref[...]


def _mid_body(s0_ref, agg_ref, deg_ref, wn_ref, ws_ref, b_ref, z_ref, s_ref):
  a = agg_ref[...]
  d = deg_ref[...]
  inv = 1.0 / jnp.maximum(d[0] + d[1], 1.0)
  h = jnp.maximum(s0_ref[...] + (a[0] + a[1]) * inv, 0.0)
  z_ref[...] = jnp.dot(h, wn_ref[...], preferred_element_type=_f32)
  s_ref[...] = jnp.dot(h, ws_ref[...], preferred_element_type=_f32) + b_ref[...]


def _fin_body(s1_ref, agg_ref, deg_ref, o_ref):
  a = agg_ref[...]
  d = deg_ref[...]
  inv = 1.0 / jnp.maximum(d[0] + d[1], 1.0)
  o_ref[...] = s1_ref[...] + (a[0] + a[1]) * inv


_row_spec = pl.BlockSpec((RB, D), lambda i: (i, 0))
_agg_spec = pl.BlockSpec((NC, RB, D), lambda i: (0, i, 0))
_deg_spec = pl.BlockSpec((NC, RB, 1), lambda i: (0, i, 0))
_w_spec = pl.BlockSpec((D, D), lambda i: (0, 0))
_b_spec = pl.BlockSpec((1, D), lambda i: (0, 0))
_nd_shape = jax.ShapeDtypeStruct((N, D), _f32)

_mm_first = pl.pallas_call(
    _first_body, grid=(GRID,),
    in_specs=[_row_spec, _w_spec, _w_spec, _b_spec],
    out_specs=[_row_spec, _row_spec],
    out_shape=[_nd_shape, _nd_shape])

_mm_mid = pl.pallas_call(
    _mid_body, grid=(GRID,),
    in_specs=[_row_spec, _agg_spec, _deg_spec, _w_spec, _w_spec, _b_spec],
    out_specs=[_row_spec, _row_spec],
    out_shape=[_nd_shape, _nd_shape])

_mm_fin = pl.pallas_call(
    _fin_body, grid=(GRID,),
    in_specs=[_row_spec, _agg_spec, _deg_spec],
    out_specs=_row_spec,
    out_shape=_nd_shape)


def _pack(idx, pad_vals):
  pk = idx[0::2] | (idx[1::2] << 16)
  pk = pk.reshape(NW, PKW)
  pad = jnp.broadcast_to(pad_vals[None, :], (NW, PKP - PKW))
  return jnp.concatenate([pk, pad], axis=1).reshape(NW, ROWS, 128)


def kernel(x, edge_index, W_self_0, W_neigh_0, b_0, W_self_1, W_neigh_1, b_1):
  spad = (jnp.arange(PKP - PKW, dtype=jnp.int32) * 79) % N
  spad = spad | (((spad * 3 + 11) % N) << 16)
  dpad = N + (jnp.arange(PKP - PKW, dtype=jnp.int32) & 63)
  dpad = dpad | (dpad << 16)
  src = _pack(edge_index[0], spad)
  dst = _pack(edge_index[1], dpad)
  b0 = b_0.reshape(1, D)
  b1 = b_1.reshape(1, D)

  z0, s0 = _mm_first(x, W_neigh_0, W_self_0, b0)
  aggp, degf = _sc_agg_deg(z0, src, dst)
  degp = degf.reshape(NC, NS * DRT)[:, :N].reshape(NC, N, 1)
  z1, s1 = _mm_mid(s0, aggp, degp, W_neigh_1, W_self_1, b1)
  (aggp1,) = _sc_agg(z1, src, dst)
  return _mm_fin(s1, aggp1, degp)


# speedup vs baseline: 7.4828x; 1.0054x over previous
import functools

import jax
import jax.numpy as jnp
from jax import lax
from jax.experimental import pallas as pl
from jax.experimental.pallas import tpu as pltpu
from jax.experimental.pallas import tpu_sc as plsc

N = 10000
D = 128
E = 320000

NC = 2
NS = 16
NW = NC * NS
L = 16
NP = 10112
RPT = NP // NS
ZB = 79
EW = E // NW
PKW = EW // 2
PKP = 5120
PADE = (PKP - PKW) * 2
B = 64
ROWS = PKP // 128
CPR = 4

DRT = 640

RB = 1000
GRID = N // RB

_f32 = jnp.float32


def _make_sc_agg(with_deg: bool):
  mesh = plsc.VectorSubcoreMesh(
      core_axis_name="c", subcore_axis_name="s",
      num_cores=NC, num_subcores=NS)
  out_type = [jax.ShapeDtypeStruct((NC, NP, D), _f32)]
  scratch = [
      pltpu.VMEM((ROWS, 128), jnp.int32),
      pltpu.VMEM((ROWS, 128), jnp.int32),
      pltpu.VMEM((4, B), jnp.int32),
      pltpu.VMEM((4, B), jnp.int32),
      pltpu.VMEM((4, B, D), _f32),
      pltpu.VMEM((8, D), _f32),
      pltpu.VMEM_SHARED((NP, D), _f32),
      pltpu.SemaphoreType.DMA((4,)),
      pltpu.SemaphoreType.DMA((4,)),
      pltpu.SemaphoreType.DMA,
  ]
  if with_deg:
    out_type.append(jax.ShapeDtypeStruct((NC * NS * DRT,), _f32))
    scratch += [
        pltpu.VMEM((B,), _f32),
        pltpu.VMEM((DRT,), _f32),
        pltpu.VMEM_SHARED((NS * DRT,), _f32),
    ]

  def body(z_hbm, src_hbm, dst_hbm, *rest):
    if with_deg:
      (agg_out, deg_out, srcp_v, dstp_v, sidx, didx, rows_v, wb8,
       agg_sh, semg, sems, semd, ones_v, degb, deg_sh) = rest
    else:
      (agg_out, srcp_v, dstp_v, sidx, didx, rows_v, wb8, agg_sh,
       semg, sems, semd) = rest

    cid = lax.axis_index("c")
    sid = lax.axis_index("s")
    w = cid * NS + sid
    base = sid * RPT
    zero16 = jnp.zeros((L,), _f32)

    def zb(i, c):
      for k in range(D // L):
        wb8[i, pl.ds(k * L, L)] = zero16
      return c
    lax.fori_loop(0, 8, zb, 0)

    def zs(t, c):
      pltpu.sync_copy(wb8, agg_sh.at[pl.ds(base + t * 8, 8)])
      return c
    lax.fori_loop(0, RPT // 8, zs, 0)
    pltpu.sync_copy(src_hbm.at[w], srcp_v)
    pltpu.sync_copy(dst_hbm.at[w], dstp_v)

    if with_deg:
      def zdg(i, c):
        degb[pl.ds(i * L, L)] = zero16
        return c
      lax.fori_loop(0, DRT // L, zdg, 0)
      pltpu.sync_copy(degb, deg_sh.at[pl.ds(sid * DRT, DRT)])
      for k in range(B // L):
        ones_v[pl.ds(k * L, L)] = jnp.ones((L,), _f32)

    plsc.subcore_barrier()

    def row(rr, c):
      copies = []
      for q in range(4):
        for g in range(2):
          spk = srcp_v[rr, pl.ds((q * 2 + g) * L, L)]
          dpk = dstp_v[rr, pl.ds((q * 2 + g) * L, L)]
          sidx[q, pl.ds(g * 2 * L, L)] = spk & 0xFFFF
          sidx[q, pl.ds((g * 2 + 1) * L, L)] = lax.shift_right_logical(spk, 16)
          didx[q, pl.ds(g * 2 * L, L)] = dpk & 0xFFFF
          didx[q, pl.ds((g * 2 + 1) * L, L)] = lax.shift_right_logical(dpk, 16)
        copies.append(pltpu.async_copy(z_hbm.at[sidx.at[q]], rows_v.at[q],
                                       semg.at[q]))
      scs, dgs = [], []
      for q in range(4):
        copies[q].wait()
        scs.append(pltpu.async_copy(rows_v.at[q], agg_sh.at[didx.at[q]],
                                    sems.at[q], add=True))
        if with_deg:
          dgs.append(pltpu.async_copy(ones_v, deg_sh.at[didx.at[q]],
                                      semd, add=True))
      for d in scs + dgs:
        d.wait()
      return c
    lax.fori_loop(0, ROWS, row, 0)

    plsc.subcore_barrier()

    def wb(t, c):
      pltpu.sync_copy(agg_sh.at[pl.ds(base + t * 8, 8)], wb8)
      pltpu.sync_copy(wb8, agg_out.at[cid, pl.ds(base + t * 8, 8)])
      return c
    lax.fori_loop(0, RPT // 8, wb, 0)
    if with_deg:
      pltpu.sync_copy(deg_sh.at[pl.ds(sid * DRT, DRT)], degb)
      pltpu.sync_copy(degb, deg_out.at[pl.ds((cid * NS + sid) * DRT, DRT)])

  return pl.kernel(body, out_type=out_type, mesh=mesh,
                   scratch_types=scratch)


_sc_agg_deg = _make_sc_agg(True)
_sc_agg = _make_sc_agg(False)


def _z_body(x_ref, wn_ref, z_ref):
  z_ref[...] = jnp.dot(x_ref[...], wn_ref[...], preferred_element_type=_f32)


def _s_body(x_ref, ws_ref, b_ref, s_ref):
  s_ref[...] = jnp.dot(x_ref[...], ws_ref[...],
                       preferred_element_type=_f32) + b_ref[...]


def _mid_body(add_bias, s0_ref, agg_ref, deg_ref, w_ref, b_ref, o_ref):
  a = agg_ref[...]
  d = deg_ref[...]
  inv = 1.0 / jnp.maximum(d[0] + d[1], 1.0)
  h = jnp.maximum(s0_ref[...] + (a[0] + a[1]) * inv, 0.0)
  o = jnp.dot(h, w_ref[...], preferred_element_type=_f32)
  o_ref[...] = o + b_ref[...] if add_bias else o


def _fin_body(s1_ref, agg_ref, deg_ref, o_ref):
  a = agg_ref[...]
  d = deg_ref[...]
  inv = 1.0 / jnp.maximum(d[0] + d[1], 1.0)
  o_ref[...] = s1_ref[...] + (a[0] + a[1]) * inv


_row_spec = pl.BlockSpec((RB, D), lambda i: (i, 0))
_agg_spec = pl.BlockSpec((NC, RB, D), lambda i: (0, i, 0))
_deg_spec = pl.BlockSpec((NC, RB, 1), lambda i: (0, i, 0))
_w_spec = pl.BlockSpec((D, D), lambda i: (0, 0))
_b_spec = pl.BlockSpec((1, D), lambda i: (0, 0))
_nd_shape = jax.ShapeDtypeStruct((N, D), _f32)

_mm_z = pl.pallas_call(
    _z_body, grid=(GRID,),
    in_specs=[_row_spec, _w_spec],
    out_specs=_row_spec, out_shape=_nd_shape)

_mm_s = pl.pallas_call(
    _s_body, grid=(GRID,),
    in_specs=[_row_spec, _w_spec, _b_spec],
    out_specs=_row_spec, out_shape=_nd_shape)

_mm_mid_z = pl.pallas_call(
    functools.partial(_mid_body, False), grid=(GRID,),
    in_specs=[_row_spec, _agg_spec, _deg_spec, _w_spec, _b_spec],
    out_specs=_row_spec, out_shape=_nd_shape)

_mm_mid_s = pl.pallas_call(
    functools.partial(_mid_body, True), grid=(GRID,),
    in_specs=[_row_spec, _agg_spec, _deg_spec, _w_spec, _b_spec],
    out_specs=_row_spec, out_shape=_nd_shape)

_mm_fin = pl.pallas_call(
    _fin_body, grid=(GRID,),
    in_specs=[_row_spec, _agg_spec, _deg_spec],
    out_specs=_row_spec,
    out_shape=_nd_shape)


def _pack(idx, pad_vals):
  pk = idx[0::2] | (idx[1::2] << 16)
  pk = pk.reshape(NW, PKW)
  pad = jnp.broadcast_to(pad_vals[None, :], (NW, PKP - PKW))
  return jnp.concatenate([pk, pad], axis=1).reshape(NW, ROWS, 128)


def kernel(x, edge_index, W_self_0, W_neigh_0, b_0, W_self_1, W_neigh_1, b_1):
  spad = (jnp.arange(PKP - PKW, dtype=jnp.int32) * 79) % N
  spad = spad | (((spad * 3 + 11) % N) << 16)
  dpad = N + (jnp.arange(PKP - PKW, dtype=jnp.int32) & 63)
  dpad = dpad | (dpad << 16)
  src = _pack(edge_index[0], spad)
  dst = _pack(edge_index[1], dpad)
  b0 = b_0.reshape(1, D)
  b1 = b_1.reshape(1, D)

  z0 = _mm_z(x, W_neigh_0)
  aggp, degf = _sc_agg_deg(z0, src, dst)
  s0 = _mm_s(x, W_self_0, b0)
  degp = degf.reshape(NC, NS * DRT)[:, :N].reshape(NC, N, 1)
  z1 = _mm_mid_z(s0, aggp, degp, W_neigh_1, b1)
  (aggp1,) = _sc_agg(z1, src, dst)
  s1 = _mm_mid_s(s0, aggp, degp, W_self_1, b1)
  return _mm_fin(s1, aggp1, degp)


# speedup vs baseline: 8.5498x; 1.1426x over previous
import functools

import jax
import jax.numpy as jnp
from jax import lax
from jax.experimental import pallas as pl
from jax.experimental.pallas import tpu as pltpu
from jax.experimental.pallas import tpu_sc as plsc

N = 10000
D = 128
E = 320000

NC = 2
NS = 16
NW = NC * NS
L = 16
NP = 10112
RPT = NP // NS
ZB = 79
EW = E // NW
PKW = EW // 2
PKP = 5120
PADE = (PKP - PKW) * 2
B = 64
ROWS = PKP // 128
CPR = 4

DRT = 640

RB = 1000
GRID = N // RB

_f32 = jnp.float32


def _make_sc_agg(with_deg: bool):
  mesh = plsc.VectorSubcoreMesh(
      core_axis_name="c", subcore_axis_name="s",
      num_cores=NC, num_subcores=NS)
  out_type = [jax.ShapeDtypeStruct((NC, NP, D), _f32)]
  scratch = [
      pltpu.VMEM((ROWS, 128), jnp.int32),
      pltpu.VMEM((ROWS, 128), jnp.int32),
      pltpu.VMEM((4, B), jnp.int32),
      pltpu.VMEM((4, B), jnp.int32),
      pltpu.VMEM((4, B, D), _f32),
      pltpu.VMEM((8, D), _f32),
      pltpu.VMEM_SHARED((NP, D), _f32),
      pltpu.SemaphoreType.DMA((4,)),
      pltpu.SemaphoreType.DMA((4,)),
      pltpu.SemaphoreType.DMA,
  ]
  if with_deg:
    out_type.append(jax.ShapeDtypeStruct((NC * NS * DRT,), _f32))
    scratch += [
        pltpu.VMEM((B,), _f32),
        pltpu.VMEM((DRT,), _f32),
        pltpu.VMEM_SHARED((NS * DRT,), _f32),
    ]

  def body(z_hbm, src_hbm, dst_hbm, *rest):
    if with_deg:
      (agg_out, deg_out, srcp_v, dstp_v, sidx, didx, rows_v, wb8,
       agg_sh, semg, sems, semd, ones_v, degb, deg_sh) = rest
    else:
      (agg_out, srcp_v, dstp_v, sidx, didx, rows_v, wb8, agg_sh,
       semg, sems, semd) = rest

    cid = lax.axis_index("c")
    sid = lax.axis_index("s")
    w = cid * NS + sid
    base = sid * RPT
    zero16 = jnp.zeros((L,), _f32)

    def zb(i, c):
      for k in range(D // L):
        wb8[i, pl.ds(k * L, L)] = zero16
      return c
    lax.fori_loop(0, 8, zb, 0)

    def zs(t, c):
      pltpu.sync_copy(wb8, agg_sh.at[pl.ds(base + t * 8, 8)])
      return c
    lax.fori_loop(0, RPT // 8, zs, 0)
    pltpu.sync_copy(src_hbm.at[w], srcp_v)
    pltpu.sync_copy(dst_hbm.at[w], dstp_v)

    if with_deg:
      def zdg(i, c):
        degb[pl.ds(i * L, L)] = zero16
        return c
      lax.fori_loop(0, DRT // L, zdg, 0)
      pltpu.sync_copy(degb, deg_sh.at[pl.ds(sid * DRT, DRT)])
      for k in range(B // L):
        ones_v[pl.ds(k * L, L)] = jnp.ones((L,), _f32)

    plsc.subcore_barrier()

    def row(rr, c):
      copies = []
      for q in range(4):
        @pl.when(rr > 0)
        def _drain(q=q):
          pltpu.make_async_copy(rows_v.at[q], agg_sh.at[didx.at[q]],
                                sems.at[q]).wait()
          if with_deg:
            pltpu.make_async_copy(ones_v, deg_sh.at[didx.at[q]],
                                  semd).wait()
        for g in range(2):
          spk = srcp_v[rr, pl.ds((q * 2 + g) * L, L)]
          dpk = dstp_v[rr, pl.ds((q * 2 + g) * L, L)]
          sidx[q, pl.ds(g * 2 * L, L)] = spk & 0xFFFF
          sidx[q, pl.ds((g * 2 + 1) * L, L)] = lax.shift_right_logical(spk, 16)
          didx[q, pl.ds(g * 2 * L, L)] = dpk & 0xFFFF
          didx[q, pl.ds((g * 2 + 1) * L, L)] = lax.shift_right_logical(dpk, 16)
        copies.append(pltpu.async_copy(z_hbm.at[sidx.at[q]], rows_v.at[q],
                                       semg.at[q]))
      for q in range(4):
        copies[q].wait()
        pltpu.async_copy(rows_v.at[q], agg_sh.at[didx.at[q]],
                         sems.at[q], add=True)
        if with_deg:
          pltpu.async_copy(ones_v, deg_sh.at[didx.at[q]], semd, add=True)
      return c
    lax.fori_loop(0, ROWS, row, 0)
    for q in range(4):
      pltpu.make_async_copy(rows_v.at[q], agg_sh.at[didx.at[q]],
                            sems.at[q]).wait()
      if with_deg:
        pltpu.make_async_copy(ones_v, deg_sh.at[didx.at[q]], semd).wait()

    plsc.subcore_barrier()

    def wb(t, c):
      pltpu.sync_copy(agg_sh.at[pl.ds(base + t * 8, 8)], wb8)
      pltpu.sync_copy(wb8, agg_out.at[cid, pl.ds(base + t * 8, 8)])
      return c
    lax.fori_loop(0, RPT // 8, wb, 0)
    if with_deg:
      pltpu.sync_copy(deg_sh.at[pl.ds(sid * DRT, DRT)], degb)
      pltpu.sync_copy(degb, deg_out.at[pl.ds((cid * NS + sid) * DRT, DRT)])

  return pl.kernel(body, out_type=out_type, mesh=mesh,
                   scratch_types=scratch)


_sc_agg_deg = _make_sc_agg(True)
_sc_agg = _make_sc_agg(False)


def _z_body(x_ref, wn_ref, z_ref):
  z_ref[...] = jnp.dot(x_ref[...], wn_ref[...], preferred_element_type=_f32)


def _s_body(x_ref, ws_ref, b_ref, s_ref):
  s_ref[...] = jnp.dot(x_ref[...], ws_ref[...],
                       preferred_element_type=_f32) + b_ref[...]


def _mid_body(add_bias, s0_ref, agg_ref, deg_ref, w_ref, b_ref, o_ref):
  a = agg_ref[...]
  d = deg_ref[...]
  inv = 1.0 / jnp.maximum(d[0] + d[1], 1.0)
  h = jnp.maximum(s0_ref[...] + (a[0] + a[1]) * inv, 0.0)
  o = jnp.dot(h, w_ref[...], preferred_element_type=_f32)
  o_ref[...] = o + b_ref[...] if add_bias else o


def _fin_body(s1_ref, agg_ref, deg_ref, o_ref):
  a = agg_ref[...]
  d = deg_ref[...]
  inv = 1.0 / jnp.maximum(d[0] + d[1], 1.0)
  o_ref[...] = s1_ref[...] + (a[0] + a[1]) * inv


_row_spec = pl.BlockSpec((RB, D), lambda i: (i, 0))
_agg_spec = pl.BlockSpec((NC, RB, D), lambda i: (0, i, 0))
_deg_spec = pl.BlockSpec((NC, RB, 1), lambda i: (0, i, 0))
_w_spec = pl.BlockSpec((D, D), lambda i: (0, 0))
_b_spec = pl.BlockSpec((1, D), lambda i: (0, 0))
_nd_shape = jax.ShapeDtypeStruct((N, D), _f32)

_mm_z = pl.pallas_call(
    _z_body, grid=(GRID,),
    in_specs=[_row_spec, _w_spec],
    out_specs=_row_spec, out_shape=_nd_shape)

_mm_s = pl.pallas_call(
    _s_body, grid=(GRID,),
    in_specs=[_row_spec, _w_spec, _b_spec],
    out_specs=_row_spec, out_shape=_nd_shape)

_mm_mid_z = pl.pallas_call(
    functools.partial(_mid_body, False), grid=(GRID,),
    in_specs=[_row_spec, _agg_spec, _deg_spec, _w_spec, _b_spec],
    out_specs=_row_spec, out_shape=_nd_shape)

_mm_mid_s = pl.pallas_call(
    functools.partial(_mid_body, True), grid=(GRID,),
    in_specs=[_row_spec, _agg_spec, _deg_spec, _w_spec, _b_spec],
    out_specs=_row_spec, out_shape=_nd_shape)

_mm_fin = pl.pallas_call(
    _fin_body, grid=(GRID,),
    in_specs=[_row_spec, _agg_spec, _deg_spec],
    out_specs=_row_spec,
    out_shape=_nd_shape)


def _pack(idx, pad_vals):
  pk = idx[0::2] | (idx[1::2] << 16)
  pk = pk.reshape(NW, PKW)
  pad = jnp.broadcast_to(pad_vals[None, :], (NW, PKP - PKW))
  return jnp.concatenate([pk, pad], axis=1).reshape(NW, ROWS, 128)


def kernel(x, edge_index, W_self_0, W_neigh_0, b_0, W_self_1, W_neigh_1, b_1):
  spad = (jnp.arange(PKP - PKW, dtype=jnp.int32) * 79) % N
  spad = spad | (((spad * 3 + 11) % N) << 16)
  dpad = N + (jnp.arange(PKP - PKW, dtype=jnp.int32) & 63)
  dpad = dpad | (dpad << 16)
  src = _pack(edge_index[0], spad)
  dst = _pack(edge_index[1], dpad)
  b0 = b_0.reshape(1, D)
  b1 = b_1.reshape(1, D)

  z0 = _mm_z(x, W_neigh_0)
  aggp, degf = _sc_agg_deg(z0, src, dst)
  s0 = _mm_s(x, W_self_0, b0)
  degp = degf.reshape(NC, NS * DRT)[:, :N].reshape(NC, N, 1)
  z1 = _mm_mid_z(s0, aggp, degp, W_neigh_1, b1)
  (aggp1,) = _sc_agg(z1, src, dst)
  s1 = _mm_mid_s(s0, aggp, degp, W_self_1, b1)
  return _mm_fin(s1, aggp1, degp)
